# Initial kernel scaffold; baseline (speedup 1.0000x reference)
import jax
import jax.numpy as jnp
from jax.experimental import pallas as pl


def kernel(x, m_ln1_g, m_ln1_b, m_attn_wqkv, m_attn_bqkv, m_attn_wo, m_attn_bo, m_ln2_g, m_ln2_b, gate_w, exp_fc_w, exp_fc_b, exp_proj_w, exp_proj_b, s_ln1_g, s_ln1_b, s_attn_wqkv, s_attn_bqkv, s_attn_wo, s_attn_bo, s_ln2_g, s_ln2_b, s_fc_w, s_fc_b, s_proj_w, s_proj_b):
    raise NotImplementedError("write your pallas kernel here")


# speedup vs baseline: 1.6324x; 1.6324x over previous
import functools

import jax
import jax.numpy as jnp
import numpy as np
from jax import lax
from jax.experimental import pallas as pl
from jax.experimental.pallas import tpu as pltpu
from jax.experimental.pallas import tpu_sc as plsc

L, D, H, E, K = 2048, 768, 12, 8, 2
DH = D // H
F = 4 * D
P = L * K
BM = 512
NT = P // BM
G = NT + E - 1
BR = 256
BQ = 512
SCALE = 1.0 / np.sqrt(DH)

_f32 = jnp.float32


def _ln(x, g, b):
    m = jnp.mean(x, axis=-1, keepdims=True)
    v = jnp.mean((x - m) ** 2, axis=-1, keepdims=True)
    return (x - m) * lax.rsqrt(v + 1e-5) * g + b


def _gelu(x):
    return x * (1.0 / (1.0 + jnp.exp(-1.702 * x)))


def _ln_qkv_body(x_ref, g_ref, b_ref, w_ref, bq_ref, qkv_ref):
    h = _ln(x_ref[...], g_ref[...], b_ref[...])
    qkv_ref[...] = (
        jnp.dot(h, w_ref[...], preferred_element_type=_f32) + bq_ref[...]
    )


def _ln_qkv(x, g, b, w_t, bq):
    return pl.pallas_call(
        _ln_qkv_body,
        grid=(L // BR,),
        in_specs=[
            pl.BlockSpec((BR, D), lambda i: (i, 0)),
            pl.BlockSpec((1, D), lambda i: (0, 0)),
            pl.BlockSpec((1, D), lambda i: (0, 0)),
            pl.BlockSpec((D, 3 * D), lambda i: (0, 0)),
            pl.BlockSpec((1, 3 * D), lambda i: (0, 0)),
        ],
        out_specs=pl.BlockSpec((BR, 3 * D), lambda i: (i, 0)),
        out_shape=jax.ShapeDtypeStruct((L, 3 * D), _f32),
    )(x, g, b, w_t, bq)


def _attn_body(q_ref, k_ref, v_ref, o_ref):
    q2 = q_ref[...] * SCALE
    k2 = k_ref[...]
    v2 = v_ref[...]
    outs = []
    for hh in range(2):
        sl = slice(hh * DH, (hh + 1) * DH)
        s = lax.dot_general(q2[:, sl], k2[:, sl], (((1,), (1,)), ((), ())),
                            preferred_element_type=_f32)
        mx = jnp.max(s, axis=-1, keepdims=True)
        p = jnp.exp(s - mx)
        p = p / jnp.sum(p, axis=-1, keepdims=True)
        outs.append(jnp.dot(p, v2[:, sl], preferred_element_type=_f32))
    o_ref[...] = jnp.concatenate(outs, axis=1)


def _attention(qkv):
    HP = H // 2
    return pl.pallas_call(
        _attn_body,
        grid=(HP, L // BQ),
        in_specs=[
            pl.BlockSpec((BQ, 2 * DH), lambda h, i: (i, h)),
            pl.BlockSpec((L, 2 * DH), lambda h, i: (0, HP + h)),
            pl.BlockSpec((L, 2 * DH), lambda h, i: (0, 2 * HP + h)),
        ],
        out_specs=pl.BlockSpec((BQ, 2 * DH), lambda h, i: (i, h)),
        out_shape=jax.ShapeDtypeStruct((L, D), _f32),
    )(qkv, qkv, qkv)


def _proj_res_body(x_ref, a_ref, w_ref, b_ref, o_ref):
    o_ref[...] = x_ref[...] + (
        jnp.dot(a_ref[...], w_ref[...], preferred_element_type=_f32)
        + b_ref[...]
    )


def _proj_residual(x, attn, wo_t, bo):
    return pl.pallas_call(
        _proj_res_body,
        grid=(L // BR,),
        in_specs=[
            pl.BlockSpec((BR, D), lambda i: (i, 0)),
            pl.BlockSpec((BR, D), lambda i: (i, 0)),
            pl.BlockSpec((D, D), lambda i: (0, 0)),
            pl.BlockSpec((1, D), lambda i: (0, 0)),
        ],
        out_specs=pl.BlockSpec((BR, D), lambda i: (i, 0)),
        out_shape=jax.ShapeDtypeStruct((L, D), _f32),
    )(x, attn, wo_t, bo)


def _router_body(x_ref, g_ref, b_ref, gw_ref, hs_ref, lg_ref, w_ref, e_ref):
    h = _ln(x_ref[...], g_ref[...], b_ref[...])
    hs_ref[...] = h
    logits = jnp.dot(h, gw_ref[...], preferred_element_type=_f32)
    lg_ref[...] = logits
    mx = jnp.max(logits, axis=-1, keepdims=True)
    ex = jnp.exp(logits - mx)
    rw = ex / jnp.sum(ex, axis=-1, keepdims=True)
    lane = lax.broadcasted_iota(jnp.int32, (BR, E), 1)
    m1 = jnp.max(rw, axis=-1, keepdims=True)
    e0 = jnp.min(jnp.where(rw == m1, lane, E), axis=-1, keepdims=True)
    rw2 = jnp.where(lane == e0, -1.0, rw)
    m2 = jnp.max(rw2, axis=-1, keepdims=True)
    e1 = jnp.min(jnp.where(rw2 == m2, lane, E), axis=-1, keepdims=True)
    tot = m1 + m2
    w_ref[...] = jnp.concatenate([m1 / tot, m2 / tot], axis=-1)
    e_ref[...] = jnp.concatenate([e0, e1], axis=-1)


def _router(x1, g, b, gw_t):
    return pl.pallas_call(
        _router_body,
        grid=(L // BR,),
        in_specs=[
            pl.BlockSpec((BR, D), lambda i: (i, 0)),
            pl.BlockSpec((1, D), lambda i: (0, 0)),
            pl.BlockSpec((1, D), lambda i: (0, 0)),
            pl.BlockSpec((D, E), lambda i: (0, 0)),
        ],
        out_specs=[
            pl.BlockSpec((BR, D), lambda i: (i, 0)),
            pl.BlockSpec((BR, E), lambda i: (i, 0)),
            pl.BlockSpec((BR, K), lambda i: (i, 0)),
            pl.BlockSpec((BR, K), lambda i: (i, 0)),
        ],
        out_shape=[
            jax.ShapeDtypeStruct((L, D), _f32),
            jax.ShapeDtypeStruct((L, E), _f32),
            jax.ShapeDtypeStruct((L, K), _f32),
            jax.ShapeDtypeStruct((L, K), jnp.int32),
        ],
    )(x1, g, b, gw_t)


def _gmm_body(tid_ref, eid_ref, vld_ref, fst_ref, off_ref,
              xs_ref, fcw_ref, fcb_ref, pjw_ref, pjb_ref, out_ref):
    g = pl.program_id(0)
    e = eid_ref[g]
    m = tid_ref[g]

    @pl.when(vld_ref[g] == 1)
    def _():
        x = xs_ref[...]
        h = jnp.dot(x, fcw_ref[0], preferred_element_type=_f32) + fcb_ref[0]
        h = _gelu(h)
        o = jnp.dot(h, pjw_ref[0], preferred_element_type=_f32) + pjb_ref[0]
        rows = m * BM + lax.broadcasted_iota(jnp.int32, (BM, 1), 0)
        mask = (rows >= off_ref[e]) & (rows < off_ref[e + 1])
        contrib = jnp.where(mask, o, 0.0)

        @pl.when(fst_ref[g] == 1)
        def _():
            out_ref[...] = contrib

        @pl.when(fst_ref[g] == 0)
        def _():
            out_ref[...] = out_ref[...] + contrib


def _grouped_ffn(xs, fcw_t, fcb, pjw_t, pjb, tile_ids, expert_ids, valid,
                 first, off):
    grid_spec = pltpu.PrefetchScalarGridSpec(
        num_scalar_prefetch=5,
        grid=(G,),
        in_specs=[
            pl.BlockSpec((BM, D), lambda g, t, e, v, f, o: (t[g], 0)),
            pl.BlockSpec((1, D, F), lambda g, t, e, v, f, o: (e[g], 0, 0)),
            pl.BlockSpec((1, 1, F), lambda g, t, e, v, f, o: (e[g], 0, 0)),
            pl.BlockSpec((1, F, D), lambda g, t, e, v, f, o: (e[g], 0, 0)),
            pl.BlockSpec((1, 1, D), lambda g, t, e, v, f, o: (e[g], 0, 0)),
        ],
        out_specs=pl.BlockSpec((BM, D), lambda g, t, e, v, f, o: (t[g], 0)),
    )
    return pl.pallas_call(
        _gmm_body,
        grid_spec=grid_spec,
        out_shape=jax.ShapeDtypeStruct((P, D), _f32),
    )(tile_ids, expert_ids, valid, first, off, xs, fcw_t, fcb, pjw_t, pjb)


def _combine_body(x1_ref, y0_ref, y1_ref, w_ref, g_ref, b_ref, wq_ref,
                  bq_ref, x2_ref, qkv_ref):
    w = w_ref[...]
    x2 = x1_ref[...] + y0_ref[...] * w[:, 0:1] + y1_ref[...] * w[:, 1:2]
    x2_ref[...] = x2
    h = _ln(x2, g_ref[...], b_ref[...])
    qkv_ref[...] = (
        jnp.dot(h, wq_ref[...], preferred_element_type=_f32) + bq_ref[...]
    )


def _combine_ln_qkv(x1, y0, y1, w01, g, b, wq_t, bq):
    return pl.pallas_call(
        _combine_body,
        grid=(L // BR,),
        in_specs=[
            pl.BlockSpec((BR, D), lambda i: (i, 0)),
            pl.BlockSpec((BR, D), lambda i: (i, 0)),
            pl.BlockSpec((BR, D), lambda i: (i, 0)),
            pl.BlockSpec((BR, K), lambda i: (i, 0)),
            pl.BlockSpec((1, D), lambda i: (0, 0)),
            pl.BlockSpec((1, D), lambda i: (0, 0)),
            pl.BlockSpec((D, 3 * D), lambda i: (0, 0)),
            pl.BlockSpec((1, 3 * D), lambda i: (0, 0)),
        ],
        out_specs=[
            pl.BlockSpec((BR, D), lambda i: (i, 0)),
            pl.BlockSpec((BR, 3 * D), lambda i: (i, 0)),
        ],
        out_shape=[
            jax.ShapeDtypeStruct((L, D), _f32),
            jax.ShapeDtypeStruct((L, 3 * D), _f32),
        ],
    )(x1, y0, y1, w01, g, b, wq_t, bq)


def _mlp_body(x_ref, g_ref, b_ref, fw_ref, fb_ref, pw_ref, pb_ref, o_ref):
    h = _ln(x_ref[...], g_ref[...], b_ref[...])
    h = _gelu(jnp.dot(h, fw_ref[...], preferred_element_type=_f32)
              + fb_ref[...])
    o_ref[...] = x_ref[...] + (
        jnp.dot(h, pw_ref[...], preferred_element_type=_f32) + pb_ref[...]
    )


def _mlp_residual(x3, g, b, fw_t, fb, pw_t, pb):
    return pl.pallas_call(
        _mlp_body,
        grid=(L // BR,),
        in_specs=[
            pl.BlockSpec((BR, D), lambda i: (i, 0)),
            pl.BlockSpec((1, D), lambda i: (0, 0)),
            pl.BlockSpec((1, D), lambda i: (0, 0)),
            pl.BlockSpec((D, F), lambda i: (0, 0)),
            pl.BlockSpec((1, F), lambda i: (0, 0)),
            pl.BlockSpec((F, D), lambda i: (0, 0)),
            pl.BlockSpec((1, D), lambda i: (0, 0)),
        ],
        out_specs=pl.BlockSpec((BR, D), lambda i: (i, 0)),
        out_shape=jax.ShapeDtypeStruct((L, D), _f32),
    )(x3, g, b, fw_t, fb, pw_t, pb)


_NC = 2
_NS = 16
_NW = _NC * _NS
_TPW = L // _NW


def _sc_scatter_sorted(hs, slot0, slot1):
    mesh = plsc.VectorSubcoreMesh(core_axis_name="c", subcore_axis_name="s")

    @functools.partial(
        pl.kernel,
        out_type=jax.ShapeDtypeStruct((P, D), _f32),
        mesh=mesh,
        scratch_types=[
            pltpu.VMEM((_TPW,), jnp.int32),
            pltpu.VMEM((_TPW,), jnp.int32),
            pltpu.VMEM((_TPW, D), _f32),
            pltpu.SemaphoreType.DMA,
        ],
    )
    def k(hs_hbm, s0_hbm, s1_hbm, out_hbm, i0_v, i1_v, rows_v, sem):
        wid = lax.axis_index("s") * _NC + lax.axis_index("c")
        base = wid * _TPW
        pltpu.sync_copy(s0_hbm.at[pl.ds(base, _TPW)], i0_v)
        pltpu.sync_copy(s1_hbm.at[pl.ds(base, _TPW)], i1_v)
        pltpu.sync_copy(hs_hbm.at[pl.ds(base, _TPW)], rows_v)
        pltpu.async_copy(rows_v, out_hbm.at[i0_v], sem).wait()
        pltpu.async_copy(rows_v, out_hbm.at[i1_v], sem).wait()

    return k(hs, slot0, slot1)


def _sc_gather_back(ys, slot0, slot1):
    mesh = plsc.VectorSubcoreMesh(core_axis_name="c", subcore_axis_name="s")

    @functools.partial(
        pl.kernel,
        out_type=(
            jax.ShapeDtypeStruct((L, D), _f32),
            jax.ShapeDtypeStruct((L, D), _f32),
        ),
        mesh=mesh,
        scratch_types=[
            pltpu.VMEM((_TPW,), jnp.int32),
            pltpu.VMEM((_TPW, D), _f32),
            pltpu.SemaphoreType.DMA,
        ],
    )
    def k(ys_hbm, s0_hbm, s1_hbm, y0_hbm, y1_hbm, i_v, rows_v, sem):
        wid = lax.axis_index("s") * _NC + lax.axis_index("c")
        base = wid * _TPW
        pltpu.sync_copy(s0_hbm.at[pl.ds(base, _TPW)], i_v)
        pltpu.async_copy(ys_hbm.at[i_v], rows_v, sem).wait()
        pltpu.sync_copy(rows_v, y0_hbm.at[pl.ds(base, _TPW)])
        pltpu.sync_copy(s1_hbm.at[pl.ds(base, _TPW)], i_v)
        pltpu.async_copy(ys_hbm.at[i_v], rows_v, sem).wait()
        pltpu.sync_copy(rows_v, y1_hbm.at[pl.ds(base, _TPW)])

    return k(ys, slot0, slot1)


def _dispatch_plan(e01):
    ef = e01.reshape(-1)
    oh = (ef[:, None] == jnp.arange(E, dtype=jnp.int32)[None, :])
    csum = jnp.cumsum(oh.astype(jnp.int32), axis=0)
    counts = csum[-1]
    off = jnp.concatenate(
        [jnp.zeros((1,), jnp.int32), jnp.cumsum(counts)]).astype(jnp.int32)
    rank = jnp.take_along_axis(csum, ef[:, None], axis=1)[:, 0] - 1
    slot = (off[ef] + rank).astype(jnp.int32)
    slot01 = slot.reshape(L, K)

    m_ids = jnp.repeat(jnp.arange(NT, dtype=jnp.int32), E)
    e_ids = jnp.tile(jnp.arange(E, dtype=jnp.int32), NT)
    lo = jnp.maximum(m_ids * BM, off[e_ids])
    hi = jnp.minimum((m_ids + 1) * BM, off[e_ids + 1])
    valid_all = hi > lo
    ordkey = jnp.where(valid_all, 0, 64) + jnp.arange(NT * E)
    order = jnp.argsort(ordkey)
    tile_all = m_ids[order]
    exp_all = e_ids[order]
    v_all = valid_all[order]
    nv = jnp.sum(valid_all.astype(jnp.int32))
    lastm = tile_all[nv - 1]
    laste = exp_all[nv - 1]
    vG = v_all[:G]
    tile_ids = jnp.where(vG, tile_all[:G], lastm).astype(jnp.int32)
    expert_ids = jnp.where(vG, exp_all[:G], laste).astype(jnp.int32)
    prev_t = jnp.concatenate([tile_ids[:1] - 1, tile_ids[:-1]])
    first = ((tile_ids != prev_t) & vG).astype(jnp.int32)
    return (slot01[:, 0], slot01[:, 1], tile_ids, expert_ids,
            vG.astype(jnp.int32), first, off)


def kernel(x, m_ln1_g, m_ln1_b, m_attn_wqkv, m_attn_bqkv, m_attn_wo,
           m_attn_bo, m_ln2_g, m_ln2_b, gate_w, exp_fc_w, exp_fc_b,
           exp_proj_w, exp_proj_b, s_ln1_g, s_ln1_b, s_attn_wqkv,
           s_attn_bqkv, s_attn_wo, s_attn_bo, s_ln2_g, s_ln2_b, s_fc_w,
           s_fc_b, s_proj_w, s_proj_b):
    xf = x.reshape(L, D)
    r1 = lambda a: a.reshape(1, -1)

    qkv1 = _ln_qkv(xf, r1(m_ln1_g), r1(m_ln1_b), m_attn_wqkv.T,
                   r1(m_attn_bqkv))
    attn1 = _attention(qkv1)
    x1 = _proj_residual(xf, attn1, m_attn_wo.T, r1(m_attn_bo))

    hs, router_logits, w01, e01 = _router(x1, r1(m_ln2_g), r1(m_ln2_b),
                                          gate_w.T)
    slot0, slot1, tile_ids, expert_ids, valid, first, off = _dispatch_plan(
        e01)

    xs = _sc_scatter_sorted(hs, slot0, slot1)
    ys = _grouped_ffn(
        xs,
        jnp.swapaxes(exp_fc_w, 1, 2),
        exp_fc_b.reshape(E, 1, F),
        jnp.swapaxes(exp_proj_w, 1, 2),
        exp_proj_b.reshape(E, 1, D),
        tile_ids, expert_ids, valid, first, off)
    y0, y1 = _sc_gather_back(ys, slot0, slot1)

    x2, qkv2 = _combine_ln_qkv(x1, y0, y1, w01, r1(s_ln1_g), r1(s_ln1_b),
                               s_attn_wqkv.T, r1(s_attn_bqkv))
    attn2 = _attention(qkv2)
    x3 = _proj_residual(x2, attn2, s_attn_wo.T, r1(s_attn_bo))
    x4 = _mlp_residual(x3, r1(s_ln2_g), r1(s_ln2_b), s_fc_w.T, r1(s_fc_b),
                       s_proj_w.T, r1(s_proj_b))
    return (x4.reshape(L, 1, D), router_logits)


# speedup vs baseline: 2.0462x; 1.2535x over previous
import functools

import jax
import jax.numpy as jnp
import numpy as np
from jax import lax
from jax.experimental import pallas as pl
from jax.experimental.pallas import tpu as pltpu
from jax.experimental.pallas import tpu_sc as plsc

L, D, H, E, K = 2048, 768, 12, 8, 2
DH = D // H
F = 4 * D
P = L * K
BM = 512
NT = P // BM
G = NT + E - 1
BR = 256
BQ = 512
SCALE = 1.0 / np.sqrt(DH)

_f32 = jnp.float32


def _ln(x, g, b):
    m = jnp.mean(x, axis=-1, keepdims=True)
    v = jnp.mean((x - m) ** 2, axis=-1, keepdims=True)
    return (x - m) * lax.rsqrt(v + 1e-5) * g + b


def _gelu(x):
    return x * (1.0 / (1.0 + jnp.exp(-1.702 * x)))


def _mm_t(a, w):
    return lax.dot_general(a, w, (((1,), (1,)), ((), ())),
                           preferred_element_type=_f32)


def _ln_qkv_body(x_ref, g_ref, b_ref, w_ref, bq_ref, qkv_ref):
    h = _ln(x_ref[...], g_ref[...], b_ref[...])
    qkv_ref[...] = _mm_t(h, w_ref[...]) + bq_ref[...]


def _ln_qkv(x, g, b, w_t, bq):
    return pl.pallas_call(
        _ln_qkv_body,
        grid=(L // BR,),
        in_specs=[
            pl.BlockSpec((BR, D), lambda i: (i, 0)),
            pl.BlockSpec((1, D), lambda i: (0, 0)),
            pl.BlockSpec((1, D), lambda i: (0, 0)),
            pl.BlockSpec((3 * D, D), lambda i: (0, 0)),
            pl.BlockSpec((1, 3 * D), lambda i: (0, 0)),
        ],
        out_specs=pl.BlockSpec((BR, 3 * D), lambda i: (i, 0)),
        out_shape=jax.ShapeDtypeStruct((L, 3 * D), _f32),
    )(x, g, b, w_t, bq)


def _attn_body(q_ref, k_ref, v_ref, o_ref):
    q2 = q_ref[...] * SCALE
    k2 = k_ref[...]
    v2 = v_ref[...]
    outs = []
    for hh in range(2):
        sl = slice(hh * DH, (hh + 1) * DH)
        s = lax.dot_general(q2[:, sl], k2[:, sl], (((1,), (1,)), ((), ())),
                            preferred_element_type=_f32)
        mx = jnp.max(s, axis=-1, keepdims=True)
        p = jnp.exp(s - mx)
        p = p / jnp.sum(p, axis=-1, keepdims=True)
        outs.append(jnp.dot(p, v2[:, sl], preferred_element_type=_f32))
    o_ref[...] = jnp.concatenate(outs, axis=1)


def _attention(qkv):
    HP = H // 2
    return pl.pallas_call(
        _attn_body,
        grid=(HP, L // BQ),
        in_specs=[
            pl.BlockSpec((BQ, 2 * DH), lambda h, i: (i, h)),
            pl.BlockSpec((L, 2 * DH), lambda h, i: (0, HP + h)),
            pl.BlockSpec((L, 2 * DH), lambda h, i: (0, 2 * HP + h)),
        ],
        out_specs=pl.BlockSpec((BQ, 2 * DH), lambda h, i: (i, h)),
        out_shape=jax.ShapeDtypeStruct((L, D), _f32),
    )(qkv, qkv, qkv)


def _proj_res_body(x_ref, a_ref, w_ref, b_ref, o_ref):
    o_ref[...] = x_ref[...] + _mm_t(a_ref[...], w_ref[...]) + b_ref[...]


def _proj_residual(x, attn, wo_t, bo):
    return pl.pallas_call(
        _proj_res_body,
        grid=(L // BR,),
        in_specs=[
            pl.BlockSpec((BR, D), lambda i: (i, 0)),
            pl.BlockSpec((BR, D), lambda i: (i, 0)),
            pl.BlockSpec((D, D), lambda i: (0, 0)),
            pl.BlockSpec((1, D), lambda i: (0, 0)),
        ],
        out_specs=pl.BlockSpec((BR, D), lambda i: (i, 0)),
        out_shape=jax.ShapeDtypeStruct((L, D), _f32),
    )(x, attn, wo_t, bo)


def _router_body(x_ref, g_ref, b_ref, gw_ref, hs_ref, lg_ref, w_ref, e_ref):
    h = _ln(x_ref[...], g_ref[...], b_ref[...])
    hs_ref[...] = h
    logits = _mm_t(h, gw_ref[...])
    lg_ref[...] = logits
    mx = jnp.max(logits, axis=-1, keepdims=True)
    ex = jnp.exp(logits - mx)
    rw = ex / jnp.sum(ex, axis=-1, keepdims=True)
    lane = lax.broadcasted_iota(jnp.int32, (BR, E), 1)
    m1 = jnp.max(rw, axis=-1, keepdims=True)
    e0 = jnp.min(jnp.where(rw == m1, lane, E), axis=-1, keepdims=True)
    rw2 = jnp.where(lane == e0, -1.0, rw)
    m2 = jnp.max(rw2, axis=-1, keepdims=True)
    e1 = jnp.min(jnp.where(rw2 == m2, lane, E), axis=-1, keepdims=True)
    tot = m1 + m2
    w_ref[...] = jnp.concatenate([m1 / tot, m2 / tot], axis=-1)
    e_ref[...] = jnp.concatenate([e0, e1], axis=-1)


def _router(x1, g, b, gw_t):
    return pl.pallas_call(
        _router_body,
        grid=(L // BR,),
        in_specs=[
            pl.BlockSpec((BR, D), lambda i: (i, 0)),
            pl.BlockSpec((1, D), lambda i: (0, 0)),
            pl.BlockSpec((1, D), lambda i: (0, 0)),
            pl.BlockSpec((E, D), lambda i: (0, 0)),
        ],
        out_specs=[
            pl.BlockSpec((BR, D), lambda i: (i, 0)),
            pl.BlockSpec((BR, E), lambda i: (i, 0)),
            pl.BlockSpec((BR, K), lambda i: (i, 0)),
            pl.BlockSpec((BR, K), lambda i: (i, 0)),
        ],
        out_shape=[
            jax.ShapeDtypeStruct((L, D), _f32),
            jax.ShapeDtypeStruct((L, E), _f32),
            jax.ShapeDtypeStruct((L, K), _f32),
            jax.ShapeDtypeStruct((L, K), jnp.int32),
        ],
    )(x1, g, b, gw_t)


def _gmm_body(tid_ref, eid_ref, vld_ref, fst_ref, off_ref,
              xs_ref, fcw_ref, fcb_ref, pjw_ref, pjb_ref, out_ref):
    g = pl.program_id(0)
    e = eid_ref[g]
    m = tid_ref[g]

    @pl.when(vld_ref[g] == 1)
    def _():
        x = xs_ref[...]
        h = _gelu(_mm_t(x, fcw_ref[0]) + fcb_ref[0])
        o = _mm_t(h, pjw_ref[0]) + pjb_ref[0]
        rows = m * BM + lax.broadcasted_iota(jnp.int32, (BM, 1), 0)
        mask = (rows >= off_ref[e]) & (rows < off_ref[e + 1])
        contrib = jnp.where(mask, o, 0.0)

        @pl.when(fst_ref[g] == 1)
        def _():
            out_ref[...] = contrib

        @pl.when(fst_ref[g] == 0)
        def _():
            out_ref[...] = out_ref[...] + contrib


def _grouped_ffn(xs, fcw_t, fcb, pjw_t, pjb, tile_ids, expert_ids, valid,
                 first, off):
    grid_spec = pltpu.PrefetchScalarGridSpec(
        num_scalar_prefetch=5,
        grid=(G,),
        in_specs=[
            pl.BlockSpec((BM, D), lambda g, t, e, v, f, o: (t[g], 0)),
            pl.BlockSpec((1, F, D), lambda g, t, e, v, f, o: (e[g], 0, 0)),
            pl.BlockSpec((1, 1, F), lambda g, t, e, v, f, o: (e[g], 0, 0)),
            pl.BlockSpec((1, D, F), lambda g, t, e, v, f, o: (e[g], 0, 0)),
            pl.BlockSpec((1, 1, D), lambda g, t, e, v, f, o: (e[g], 0, 0)),
        ],
        out_specs=pl.BlockSpec((BM, D), lambda g, t, e, v, f, o: (t[g], 0)),
    )
    return pl.pallas_call(
        _gmm_body,
        grid_spec=grid_spec,
        out_shape=jax.ShapeDtypeStruct((P, D), _f32),
    )(tile_ids, expert_ids, valid, first, off, xs, fcw_t, fcb, pjw_t, pjb)


def _combine_body(x1_ref, y0_ref, y1_ref, w_ref, g_ref, b_ref, wq_ref,
                  bq_ref, x2_ref, qkv_ref):
    w = w_ref[...]
    x2 = x1_ref[...] + y0_ref[...] * w[:, 0:1] + y1_ref[...] * w[:, 1:2]
    x2_ref[...] = x2
    h = _ln(x2, g_ref[...], b_ref[...])
    qkv_ref[...] = _mm_t(h, wq_ref[...]) + bq_ref[...]


def _combine_ln_qkv(x1, y0, y1, w01, g, b, wq_t, bq):
    return pl.pallas_call(
        _combine_body,
        grid=(L // BR,),
        in_specs=[
            pl.BlockSpec((BR, D), lambda i: (i, 0)),
            pl.BlockSpec((BR, D), lambda i: (i, 0)),
            pl.BlockSpec((BR, D), lambda i: (i, 0)),
            pl.BlockSpec((BR, K), lambda i: (i, 0)),
            pl.BlockSpec((1, D), lambda i: (0, 0)),
            pl.BlockSpec((1, D), lambda i: (0, 0)),
            pl.BlockSpec((3 * D, D), lambda i: (0, 0)),
            pl.BlockSpec((1, 3 * D), lambda i: (0, 0)),
        ],
        out_specs=[
            pl.BlockSpec((BR, D), lambda i: (i, 0)),
            pl.BlockSpec((BR, 3 * D), lambda i: (i, 0)),
        ],
        out_shape=[
            jax.ShapeDtypeStruct((L, D), _f32),
            jax.ShapeDtypeStruct((L, 3 * D), _f32),
        ],
    )(x1, y0, y1, w01, g, b, wq_t, bq)


def _mlp_body(x_ref, g_ref, b_ref, fw_ref, fb_ref, pw_ref, pb_ref, o_ref):
    h = _ln(x_ref[...], g_ref[...], b_ref[...])
    h = _gelu(_mm_t(h, fw_ref[...]) + fb_ref[...])
    o_ref[...] = x_ref[...] + _mm_t(h, pw_ref[...]) + pb_ref[...]


def _mlp_residual(x3, g, b, fw_t, fb, pw_t, pb):
    return pl.pallas_call(
        _mlp_body,
        grid=(L // BR,),
        in_specs=[
            pl.BlockSpec((BR, D), lambda i: (i, 0)),
            pl.BlockSpec((1, D), lambda i: (0, 0)),
            pl.BlockSpec((1, D), lambda i: (0, 0)),
            pl.BlockSpec((F, D), lambda i: (0, 0)),
            pl.BlockSpec((1, F), lambda i: (0, 0)),
            pl.BlockSpec((D, F), lambda i: (0, 0)),
            pl.BlockSpec((1, D), lambda i: (0, 0)),
        ],
        out_specs=pl.BlockSpec((BR, D), lambda i: (i, 0)),
        out_shape=jax.ShapeDtypeStruct((L, D), _f32),
    )(x3, g, b, fw_t, fb, pw_t, pb)


_NC = 2
_NS = 16
_NW = _NC * _NS
_TPW = L // _NW


def _sc_scatter_sorted(hs, slot0, slot1):
    mesh = plsc.VectorSubcoreMesh(core_axis_name="c", subcore_axis_name="s")

    @functools.partial(
        pl.kernel,
        out_type=jax.ShapeDtypeStruct((P, D), _f32),
        mesh=mesh,
        scratch_types=[
            pltpu.VMEM((_TPW,), jnp.int32),
            pltpu.VMEM((_TPW,), jnp.int32),
            pltpu.VMEM((_TPW, D), _f32),
            pltpu.SemaphoreType.DMA,
        ],
    )
    def k(hs_hbm, s0_hbm, s1_hbm, out_hbm, i0_v, i1_v, rows_v, sem):
        wid = lax.axis_index("s") * _NC + lax.axis_index("c")
        base = wid * _TPW
        pltpu.sync_copy(s0_hbm.at[pl.ds(base, _TPW)], i0_v)
        pltpu.sync_copy(s1_hbm.at[pl.ds(base, _TPW)], i1_v)
        pltpu.sync_copy(hs_hbm.at[pl.ds(base, _TPW)], rows_v)
        pltpu.async_copy(rows_v, out_hbm.at[i0_v], sem).wait()
        pltpu.async_copy(rows_v, out_hbm.at[i1_v], sem).wait()

    return k(hs, slot0, slot1)


def _sc_gather_back(ys, slot0, slot1):
    mesh = plsc.VectorSubcoreMesh(core_axis_name="c", subcore_axis_name="s")

    @functools.partial(
        pl.kernel,
        out_type=(
            jax.ShapeDtypeStruct((L, D), _f32),
            jax.ShapeDtypeStruct((L, D), _f32),
        ),
        mesh=mesh,
        scratch_types=[
            pltpu.VMEM((_TPW,), jnp.int32),
            pltpu.VMEM((_TPW, D), _f32),
            pltpu.SemaphoreType.DMA,
        ],
    )
    def k(ys_hbm, s0_hbm, s1_hbm, y0_hbm, y1_hbm, i_v, rows_v, sem):
        wid = lax.axis_index("s") * _NC + lax.axis_index("c")
        base = wid * _TPW
        pltpu.sync_copy(s0_hbm.at[pl.ds(base, _TPW)], i_v)
        pltpu.async_copy(ys_hbm.at[i_v], rows_v, sem).wait()
        pltpu.sync_copy(rows_v, y0_hbm.at[pl.ds(base, _TPW)])
        pltpu.sync_copy(s1_hbm.at[pl.ds(base, _TPW)], i_v)
        pltpu.async_copy(ys_hbm.at[i_v], rows_v, sem).wait()
        pltpu.sync_copy(rows_v, y1_hbm.at[pl.ds(base, _TPW)])

    return k(ys, slot0, slot1)


def _dispatch_plan(e01):
    ef = e01.reshape(-1)
    oh = (ef[:, None] == jnp.arange(E, dtype=jnp.int32)[None, :])
    csum = jnp.cumsum(oh.astype(jnp.int32), axis=0)
    counts = csum[-1]
    off = jnp.concatenate(
        [jnp.zeros((1,), jnp.int32), jnp.cumsum(counts)]).astype(jnp.int32)
    rank = jnp.take_along_axis(csum, ef[:, None], axis=1)[:, 0] - 1
    slot = (off[ef] + rank).astype(jnp.int32)
    slot01 = slot.reshape(L, K)

    m_ids = jnp.repeat(jnp.arange(NT, dtype=jnp.int32), E)
    e_ids = jnp.tile(jnp.arange(E, dtype=jnp.int32), NT)
    lo = jnp.maximum(m_ids * BM, off[e_ids])
    hi = jnp.minimum((m_ids + 1) * BM, off[e_ids + 1])
    valid_all = hi > lo
    ordkey = jnp.where(valid_all, 0, 64) + jnp.arange(NT * E)
    order = jnp.argsort(ordkey)
    tile_all = m_ids[order]
    exp_all = e_ids[order]
    v_all = valid_all[order]
    nv = jnp.sum(valid_all.astype(jnp.int32))
    lastm = tile_all[nv - 1]
    laste = exp_all[nv - 1]
    vG = v_all[:G]
    tile_ids = jnp.where(vG, tile_all[:G], lastm).astype(jnp.int32)
    expert_ids = jnp.where(vG, exp_all[:G], laste).astype(jnp.int32)
    prev_t = jnp.concatenate([tile_ids[:1] - 1, tile_ids[:-1]])
    first = ((tile_ids != prev_t) & vG).astype(jnp.int32)
    return (slot01[:, 0], slot01[:, 1], tile_ids, expert_ids,
            vG.astype(jnp.int32), first, off)


def kernel(x, m_ln1_g, m_ln1_b, m_attn_wqkv, m_attn_bqkv, m_attn_wo,
           m_attn_bo, m_ln2_g, m_ln2_b, gate_w, exp_fc_w, exp_fc_b,
           exp_proj_w, exp_proj_b, s_ln1_g, s_ln1_b, s_attn_wqkv,
           s_attn_bqkv, s_attn_wo, s_attn_bo, s_ln2_g, s_ln2_b, s_fc_w,
           s_fc_b, s_proj_w, s_proj_b):
    xf = x.reshape(L, D)
    r1 = lambda a: a.reshape(1, -1)

    qkv1 = _ln_qkv(xf, r1(m_ln1_g), r1(m_ln1_b), m_attn_wqkv,
                   r1(m_attn_bqkv))
    attn1 = _attention(qkv1)
    x1 = _proj_residual(xf, attn1, m_attn_wo, r1(m_attn_bo))

    hs, router_logits, w01, e01 = _router(x1, r1(m_ln2_g), r1(m_ln2_b),
                                          gate_w)
    slot0, slot1, tile_ids, expert_ids, valid, first, off = _dispatch_plan(
        e01)

    xs = _sc_scatter_sorted(hs, slot0, slot1)
    ys = _grouped_ffn(
        xs,
        exp_fc_w,
        exp_fc_b.reshape(E, 1, F),
        exp_proj_w,
        exp_proj_b.reshape(E, 1, D),
        tile_ids, expert_ids, valid, first, off)
    y0, y1 = _sc_gather_back(ys, slot0, slot1)

    x2, qkv2 = _combine_ln_qkv(x1, y0, y1, w01, r1(s_ln1_g), r1(s_ln1_b),
                               s_attn_wqkv, r1(s_attn_bqkv))
    attn2 = _attention(qkv2)
    x3 = _proj_residual(x2, attn2, s_attn_wo, r1(s_attn_bo))
    x4 = _mlp_residual(x3, r1(s_ln2_g), r1(s_ln2_b), s_fc_w, r1(s_fc_b),
                       s_proj_w, r1(s_proj_b))
    return (x4.reshape(L, 1, D), router_logits)
